# baseline (speedup 1.0000x reference)
import jax
import jax.numpy as jnp
from jax import lax
from jax.experimental import pallas as pl
from jax.experimental.pallas import tpu as pltpu
from jax.experimental.pallas import tpu_sc as plsc

N = 10000
E = 320000
U = 128
CHUNK = 96
NW = 32
NCHUNK = (E // NW + CHUNK - 1) // CHUNK
EW = NCHUNK * CHUNK
E_PAD = EW * NW
N_PAD = 10112
ROWS_PER_TILE = N_PAD // 16
NBUF = 2
UNR = 8


def _sc_segsum(tbl_stack, T, packed2d, ewb, zeros):
    mesh = plsc.VectorSubcoreMesh(core_axis_name="c", subcore_axis_name="s")
    NQ = NCHUNK // NBUF
    TAIL = NQ * NBUF

    def body(tbl, pk_hbm, ew_hbm, zeros_hbm, out, *refs):
        pk_v, ew_v = refs[0:2]
        srcu = refs[2:2 + NBUF]
        dstu = refs[2 + NBUF:2 + 2 * NBUF]
        rows = refs[2 + 2 * NBUF:2 + 3 * NBUF]
        acc = refs[2 + 3 * NBUF]
        semst = refs[3 + 3 * NBUF]
        semg = refs[4 + 3 * NBUF:4 + 4 * NBUF]
        sems = refs[4 + 4 * NBUF:4 + 5 * NBUF]
        c = lax.axis_index("c")
        s = lax.axis_index("s")
        gw = c * 16 + s

        pltpu.async_copy(pk_hbm.at[pl.ds(gw * EW, EW)], pk_v, semst)
        pltpu.async_copy(ew_hbm.at[pl.ds(gw * EW, EW)], ew_v, semst)
        pltpu.make_async_copy(pk_hbm.at[pl.ds(0, EW)], pk_v, semst).wait()
        pltpu.make_async_copy(ew_hbm.at[pl.ds(0, EW)], ew_v, semst).wait()

        def unpack(t, j, b):
            for g in range(CHUNK // 16):
                sl = pl.ds(g * 16, 16)
                p = pk_v[pl.ds(j * CHUNK + g * 16, 16)]
                srcu[b][sl] = (p & 0xFFFF) + t * N
                dstu[b][sl] = lax.shift_right_logical(p, 16)

        def issue(t, j, b):
            unpack(t, j, b)
            pltpu.async_copy(tbl.at[srcu[b]], rows[b], semg[b])

        def scale(rref, j):
            def grp(g, carry):
                for k in range(UNR):
                    e = g * UNR + k
                    w16 = plsc.load_gather(
                        ew_v, [jnp.broadcast_to(j * CHUNK + e, (16,))])
                    for cg in range(U // 16):
                        sl = pl.ds(cg * 16, 16)
                        rref[e, sl] = rref[e, sl] * w16
                return carry

            lax.fori_loop(0, CHUNK // UNR, grp, 0)

        def one_pass(t, carry):
            pltpu.sync_copy(zeros_hbm, acc.at[pl.ds(s * ROWS_PER_TILE, ROWS_PER_TILE)])
            plsc.subcore_barrier()

            for b in range(NBUF):
                issue(t, b, b)

            def quad_body(jj, carry2):
                j0 = jj * NBUF
                for b in range(NBUF):
                    pltpu.make_async_copy(tbl.at[srcu[b]], rows[b], semg[b]).wait()
                    scale(rows[b], j0 + b)
                    pltpu.async_copy(rows[b], acc.at[dstu[b]], sems[b], add=True)

                @pl.when(jj < NQ - 1)
                def _():
                    for b in range(NBUF):
                        pltpu.make_async_copy(rows[b], acc.at[dstu[b]], sems[b]).wait()
                        issue(t, j0 + b + NBUF, b)

                return carry2

            lax.fori_loop(0, NQ, quad_body, 0)
            for b in range(NBUF):
                pltpu.make_async_copy(rows[b], acc.at[dstu[b]], sems[b]).wait()
            for j in range(TAIL, NCHUNK):
                unpack(t, j, 0)
                pltpu.async_copy(tbl.at[srcu[0]], rows[0], semg[0]).wait()
                scale(rows[0], j)
                pltpu.sync_copy(rows[0], acc.at[dstu[0]], add=True)
            plsc.subcore_barrier()
            pltpu.sync_copy(acc.at[pl.ds(s * ROWS_PER_TILE, ROWS_PER_TILE)],
                            out.at[t, c, pl.ds(s * ROWS_PER_TILE, ROWS_PER_TILE)])
            plsc.subcore_barrier()
            return carry

        lax.fori_loop(0, T, one_pass, 0)

    call = pl.kernel(
        body,
        out_type=jax.ShapeDtypeStruct((T, 2, N_PAD, U), jnp.float32),
        mesh=mesh,
        scratch_types=(
            [pltpu.VMEM((EW,), jnp.int32),
             pltpu.VMEM((EW,), jnp.float32)]
            + [pltpu.VMEM((CHUNK,), jnp.int32) for _ in range(2 * NBUF)]
            + [pltpu.VMEM((CHUNK, U), jnp.float32) for _ in range(NBUF)]
            + [pltpu.VMEM_SHARED((N_PAD, U), jnp.float32)]
            + [pltpu.SemaphoreType.DMA for _ in range(1 + 2 * NBUF)]
        ),
        compiler_params=pltpu.CompilerParams(needs_layout_passes=False),
    )
    return call(tbl_stack, packed2d, ewb, zeros)


BR = 1000


def _tc1_body(X, H, P, Wru, Wcx, Bru, Bc, RHo, Uo, Qo):
    for i in range(3):
        x = X[i]
        h = H[i]
        sx = P[2 * i, 0] + P[2 * i, 1]
        sh = P[2 * i + 1, 0] + P[2 * i + 1, 1]
        cat = jnp.concatenate([x, h, sx, sh], axis=1)
        ru = jax.nn.sigmoid(jnp.dot(cat, Wru[i], preferred_element_type=jnp.float32)
                            + Bru[i])
        r = ru[:, :U]
        u = ru[:, U:]
        RHo[i] = r * h
        Uo[i] = u
        catq = jnp.concatenate([x, sx], axis=1)
        Qo[i] = jnp.dot(catq, Wcx[i], preferred_element_type=jnp.float32) + Bc[i]


def _tc1(X, H, P, Wru, Wcx, Bru, Bc):
    nb = N // BR
    blk_cell = pl.BlockSpec((3, BR, U), lambda i: (0, i, 0))
    out_row = pl.BlockSpec((BR, U), lambda i: (i, 0))
    return pl.pallas_call(
        _tc1_body,
        grid=(nb,),
        in_specs=[
            blk_cell,
            blk_cell,
            pl.BlockSpec((6, 2, BR, U), lambda i: (0, 0, i, 0)),
            pl.BlockSpec((3, 4 * U, 2 * U), lambda i: (0, 0, 0)),
            pl.BlockSpec((3, 2 * U, U), lambda i: (0, 0, 0)),
            pl.BlockSpec((3, 1, 2 * U), lambda i: (0, 0, 0)),
            pl.BlockSpec((3, 1, U), lambda i: (0, 0, 0)),
        ],
        out_specs=[blk_cell, blk_cell, blk_cell],
        out_shape=[
            jax.ShapeDtypeStruct((3, N, U), jnp.float32),
            jax.ShapeDtypeStruct((3, N, U), jnp.float32),
            jax.ShapeDtypeStruct((3, N, U), jnp.float32),
        ],
    )(X, H, P, Wru, Wcx, Bru, Bc)


def _tc2_body(Uu, Q, RH, T2, H, Wch, hf, hl, hs):
    outs = (hf, hl, hs)
    for i in range(3):
        t = T2[i, 0] + T2[i, 1]
        cat = jnp.concatenate([RH[i], t], axis=1)
        c = jnp.tanh(Q[i] + jnp.dot(cat, Wch[i], preferred_element_type=jnp.float32))
        u = Uu[i]
        outs[i][...] = u * H[i] + (1.0 - u) * c


def _tc2(Uu, Q, RH, T2, H, Wch):
    nb = N // BR
    blk_cell = pl.BlockSpec((3, BR, U), lambda i: (0, i, 0))
    out_row = pl.BlockSpec((BR, U), lambda i: (i, 0))
    return pl.pallas_call(
        _tc2_body,
        grid=(nb,),
        in_specs=[
            blk_cell,
            blk_cell,
            blk_cell,
            pl.BlockSpec((3, 2, BR, U), lambda i: (0, 0, i, 0)),
            blk_cell,
            pl.BlockSpec((3, 2 * U, U), lambda i: (0, 0, 0)),
        ],
        out_specs=[out_row, out_row, out_row],
        out_shape=[jax.ShapeDtypeStruct((N, U), jnp.float32)] * 3,
    )(Uu, Q, RH, T2, H, Wch)


def _fuse1_body(G, Wh, Wo, bh, bo, Ml, Ms):
    g = G[...]
    Ml[...] = jax.nn.sigmoid(jnp.dot(Wh[...], g, preferred_element_type=jnp.float32) + bh[...])
    Ms[...] = jax.nn.sigmoid(jnp.dot(Wo[...], g, preferred_element_type=jnp.float32) + bo[...])


def _fuse1(G, Wh, Wo, bh, bo):
    return pl.pallas_call(
        _fuse1_body,
        out_shape=[jax.ShapeDtypeStruct((U, N), jnp.float32)] * 2,
    )(G, Wh, Wo, bh, bo)


def _fuse2_body(hf, hl, hs, lw, sw, Eo):
    Eo[...] = hf[...] + lw[...] * hl[...] + sw[...] * hs[...]


def _fuse2(hf, hl, hs, lw, sw):
    nb = N // BR
    row = pl.BlockSpec((BR, U), lambda i: (i, 0))
    return pl.pallas_call(
        _fuse2_body,
        grid=(nb,),
        in_specs=[row] * 5,
        out_specs=row,
        out_shape=jax.ShapeDtypeStruct((N, U), jnp.float32),
    )(hf, hl, hs, lw, sw)


def kernel(x_od, history, yesterday, finished_hidden, long_his_hidden,
           short_his_hidden, edge_index, edge_attr, W_ru_fin, b_ru_fin,
           W_c_fin, b_c_fin, W_ru_long, b_ru_long, W_c_long, b_c_long,
           W_ru_short, b_ru_short, W_c_short, b_c_short, W_hid, b_hid,
           W_out, b_out):
    packed = edge_index[0] | (edge_index[1] << 16)
    pad_word = (N_PAD - 1) << 16
    packed1d = jnp.concatenate(
        [packed, jnp.full((E_PAD - E,), pad_word, jnp.int32)])
    ewp = jnp.concatenate([edge_attr, jnp.zeros((E_PAD - E,), jnp.float32)])
    zeros = jnp.zeros((ROWS_PER_TILE, U), jnp.float32)

    stack1 = jnp.concatenate(
        [x_od, finished_hidden, history, long_his_hidden, yesterday,
         short_his_hidden], axis=0)
    P = _sc_segsum(stack1, 6, packed1d, ewp, zeros)

    X = jnp.stack([x_od, history, yesterday])
    H = jnp.stack([finished_hidden, long_his_hidden, short_his_hidden])
    Wru = jnp.stack([W_ru_fin.reshape(2 * 2 * U, 2 * U),
                     W_ru_long.reshape(2 * 2 * U, 2 * U),
                     W_ru_short.reshape(2 * 2 * U, 2 * U)])
    Wcx = jnp.stack([W_c_fin[:, :U, :].reshape(2 * U, U),
                     W_c_long[:, :U, :].reshape(2 * U, U),
                     W_c_short[:, :U, :].reshape(2 * U, U)])
    Wch = jnp.stack([W_c_fin[:, U:, :].reshape(2 * U, U),
                     W_c_long[:, U:, :].reshape(2 * U, U),
                     W_c_short[:, U:, :].reshape(2 * U, U)])
    Bru = jnp.stack([b_ru_fin, b_ru_long, b_ru_short])[:, None, :]
    Bc = jnp.stack([b_c_fin, b_c_long, b_c_short])[:, None, :]

    RH, Uu, Q = _tc1(X, H, P, Wru, Wcx, Bru, Bc)

    T2 = _sc_segsum(RH.reshape(3 * N, U), 3, packed1d, ewp, zeros)

    hf, hl, hs = _tc2(Uu, Q, RH, T2, H, Wch)

    G = jnp.concatenate([hl, hs], axis=1).reshape(2 * U, N)
    Ml, Ms = _fuse1(G, W_hid, W_out, b_hid.reshape(U, 1), b_out.reshape(U, 1))
    lw = Ml.reshape(N, U)
    sw = Ms.reshape(N, U)

    Eo = _fuse2(hf, hl, hs, lw, sw)
    return (Eo, hf, hl, hs, Eo)


# speedup vs baseline: 1.5378x; 1.5378x over previous
import jax
import jax.numpy as jnp
from jax import lax
from jax.experimental import pallas as pl
from jax.experimental.pallas import tpu as pltpu
from jax.experimental.pallas import tpu_sc as plsc

N = 10000
E = 320000
U = 128
CHUNK = 96
NW = 32
NCHUNK = (E // NW + CHUNK - 1) // CHUNK
EW = NCHUNK * CHUNK
E_PAD = EW * NW
N_PAD = 10112
ROWS_PER_TILE = N_PAD // 16
NBUF = 2
UNR = 8


def _sc_segsum(tbl_stack, T, packed2d, ewb, zeros):
    mesh = plsc.VectorSubcoreMesh(core_axis_name="c", subcore_axis_name="s")
    NQ = NCHUNK // NBUF
    TAIL = NQ * NBUF

    def body(tbl, pk_hbm, ew_hbm, zeros_hbm, out, *refs):
        pk_v, ew_v = refs[0:2]
        srcu = refs[2:2 + NBUF]
        dstu = refs[2 + NBUF:2 + 2 * NBUF]
        rows = refs[2 + 2 * NBUF:2 + 3 * NBUF]
        acc = refs[2 + 3 * NBUF]
        semst = refs[3 + 3 * NBUF]
        semg = refs[4 + 3 * NBUF:4 + 4 * NBUF]
        sems = refs[4 + 4 * NBUF:4 + 5 * NBUF]
        c = lax.axis_index("c")
        s = lax.axis_index("s")
        gw = c * 16 + s

        pltpu.async_copy(pk_hbm.at[pl.ds(gw * EW, EW)], pk_v, semst)
        pltpu.async_copy(ew_hbm.at[pl.ds(gw * EW, EW)], ew_v, semst)
        pltpu.make_async_copy(pk_hbm.at[pl.ds(0, EW)], pk_v, semst).wait()
        pltpu.make_async_copy(ew_hbm.at[pl.ds(0, EW)], ew_v, semst).wait()

        def unpack(t, j, b):
            for g in range(CHUNK // 16):
                sl = pl.ds(g * 16, 16)
                p = pk_v[pl.ds(j * CHUNK + g * 16, 16)]
                srcu[b][sl] = (p & 0xFFFF) + t * N
                dstu[b][sl] = lax.shift_right_logical(p, 16)

        def issue(t, j, b):
            unpack(t, j, b)
            pltpu.async_copy(tbl.at[srcu[b]], rows[b], semg[b])

        def scale(rref, j):
            def grp(g, carry):
                for k in range(UNR):
                    e = g * UNR + k
                    w16 = plsc.load_gather(
                        ew_v, [jnp.broadcast_to(j * CHUNK + e, (16,))])
                    for cg in range(U // 16):
                        sl = pl.ds(cg * 16, 16)
                        rref[e, sl] = rref[e, sl] * w16
                return carry

            lax.fori_loop(0, CHUNK // UNR, grp, 0)

        def one_pass(t, carry):
            pltpu.sync_copy(zeros_hbm, acc.at[pl.ds(s * ROWS_PER_TILE, ROWS_PER_TILE)])
            plsc.subcore_barrier()

            for b in range(NBUF):
                issue(t, b, b)

            def quad_body(jj, carry2):
                j0 = jj * NBUF
                for b in range(NBUF):
                    pltpu.make_async_copy(tbl.at[srcu[b]], rows[b], semg[b]).wait()
                    scale(rows[b], j0 + b)
                    pltpu.async_copy(rows[b], acc.at[dstu[b]], sems[b], add=True)

                @pl.when(jj < NQ - 1)
                def _():
                    for b in range(NBUF):
                        pltpu.make_async_copy(rows[b], acc.at[dstu[b]], sems[b]).wait()
                        issue(t, j0 + b + NBUF, b)

                return carry2

            lax.fori_loop(0, NQ, quad_body, 0)
            for b in range(NBUF):
                pltpu.make_async_copy(rows[b], acc.at[dstu[b]], sems[b]).wait()
            for j in range(TAIL, NCHUNK):
                unpack(t, j, 0)
                pltpu.async_copy(tbl.at[srcu[0]], rows[0], semg[0]).wait()
                scale(rows[0], j)
                pltpu.sync_copy(rows[0], acc.at[dstu[0]], add=True)
            plsc.subcore_barrier()
            pltpu.sync_copy(acc.at[pl.ds(s * ROWS_PER_TILE, ROWS_PER_TILE)],
                            out.at[t, c, pl.ds(s * ROWS_PER_TILE, ROWS_PER_TILE)])
            plsc.subcore_barrier()
            return carry

        lax.fori_loop(0, T, one_pass, 0)

    call = pl.kernel(
        body,
        out_type=jax.ShapeDtypeStruct((T, 2, N_PAD, U), jnp.float32),
        mesh=mesh,
        scratch_types=(
            [pltpu.VMEM((EW,), jnp.int32),
             pltpu.VMEM((EW,), jnp.float32)]
            + [pltpu.VMEM((CHUNK,), jnp.int32) for _ in range(2 * NBUF)]
            + [pltpu.VMEM((CHUNK, U), jnp.float32) for _ in range(NBUF)]
            + [pltpu.VMEM_SHARED((N_PAD, U), jnp.float32)]
            + [pltpu.SemaphoreType.DMA for _ in range(1 + 2 * NBUF)]
        ),
        compiler_params=pltpu.CompilerParams(needs_layout_passes=False),
    )
    return call(tbl_stack, packed2d, ewb, zeros)


BR = 1000


def _tc1_body(X, H, P, Wru, Wcx, Bru, Bc, RHo, Uo, Qo):
    for i in range(3):
        x = X[i]
        h = H[i]
        sx = P[2 * i, 0] + P[2 * i, 1]
        sh = P[2 * i + 1, 0] + P[2 * i + 1, 1]
        cat = jnp.concatenate([x, h, sx, sh], axis=1)
        ru = jax.nn.sigmoid(jnp.dot(cat, Wru[i], preferred_element_type=jnp.float32)
                            + Bru[i])
        r = ru[:, :U]
        u = ru[:, U:]
        RHo[i] = r * h
        Uo[i] = u
        catq = jnp.concatenate([x, sx], axis=1)
        Qo[i] = jnp.dot(catq, Wcx[i], preferred_element_type=jnp.float32) + Bc[i]


def _tc1(X, H, P, Wru, Wcx, Bru, Bc):
    nb = N // BR
    blk_cell = pl.BlockSpec((3, BR, U), lambda i: (0, i, 0))
    out_row = pl.BlockSpec((BR, U), lambda i: (i, 0))
    return pl.pallas_call(
        _tc1_body,
        grid=(nb,),
        in_specs=[
            blk_cell,
            blk_cell,
            pl.BlockSpec((6, 2, BR, U), lambda i: (0, 0, i, 0)),
            pl.BlockSpec((3, 4 * U, 2 * U), lambda i: (0, 0, 0)),
            pl.BlockSpec((3, 2 * U, U), lambda i: (0, 0, 0)),
            pl.BlockSpec((3, 1, 2 * U), lambda i: (0, 0, 0)),
            pl.BlockSpec((3, 1, U), lambda i: (0, 0, 0)),
        ],
        out_specs=[blk_cell, blk_cell, blk_cell],
        out_shape=[
            jax.ShapeDtypeStruct((3, N, U), jnp.float32),
            jax.ShapeDtypeStruct((3, N, U), jnp.float32),
            jax.ShapeDtypeStruct((3, N, U), jnp.float32),
        ],
    )(X, H, P, Wru, Wcx, Bru, Bc)


def _tc2_body(Uu, Q, RH, T2, H, Wch, hf, hl, hs):
    outs = (hf, hl, hs)
    for i in range(3):
        t = T2[i, 0] + T2[i, 1]
        cat = jnp.concatenate([RH[i], t], axis=1)
        c = jnp.tanh(Q[i] + jnp.dot(cat, Wch[i], preferred_element_type=jnp.float32))
        u = Uu[i]
        outs[i][...] = u * H[i] + (1.0 - u) * c


def _tc2(Uu, Q, RH, T2, H, Wch):
    nb = N // BR
    blk_cell = pl.BlockSpec((3, BR, U), lambda i: (0, i, 0))
    out_row = pl.BlockSpec((BR, U), lambda i: (i, 0))
    return pl.pallas_call(
        _tc2_body,
        grid=(nb,),
        in_specs=[
            blk_cell,
            blk_cell,
            blk_cell,
            pl.BlockSpec((3, 2, BR, U), lambda i: (0, 0, i, 0)),
            blk_cell,
            pl.BlockSpec((3, 2 * U, U), lambda i: (0, 0, 0)),
        ],
        out_specs=[out_row, out_row, out_row],
        out_shape=[jax.ShapeDtypeStruct((N, U), jnp.float32)] * 3,
    )(Uu, Q, RH, T2, H, Wch)


def _fuse1_body(G, Wh, Wo, bh, bo, Ml, Ms):
    g = G[...]
    Ml[...] = jax.nn.sigmoid(jnp.dot(Wh[...], g, preferred_element_type=jnp.float32) + bh[...])
    Ms[...] = jax.nn.sigmoid(jnp.dot(Wo[...], g, preferred_element_type=jnp.float32) + bo[...])


def _fuse1(G, Wh, Wo, bh, bo):
    return pl.pallas_call(
        _fuse1_body,
        out_shape=[jax.ShapeDtypeStruct((U, N), jnp.float32)] * 2,
    )(G, Wh, Wo, bh, bo)


def _fuse2_body(hf, hl, hs, lw, sw, Eo):
    Eo[...] = hf[...] + lw[...] * hl[...] + sw[...] * hs[...]


def _fuse2(hf, hl, hs, lw, sw):
    nb = N // BR
    row = pl.BlockSpec((BR, U), lambda i: (i, 0))
    return pl.pallas_call(
        _fuse2_body,
        grid=(nb,),
        in_specs=[row] * 5,
        out_specs=row,
        out_shape=jax.ShapeDtypeStruct((N, U), jnp.float32),
    )(hf, hl, hs, lw, sw)


def kernel(x_od, history, yesterday, finished_hidden, long_his_hidden,
           short_his_hidden, edge_index, edge_attr, W_ru_fin, b_ru_fin,
           W_c_fin, b_c_fin, W_ru_long, b_ru_long, W_c_long, b_c_long,
           W_ru_short, b_ru_short, W_c_short, b_c_short, W_hid, b_hid,
           W_out, b_out):
    packed = edge_index[0] | (edge_index[1] << 16)
    pad_idx = jnp.arange(E_PAD - E, dtype=jnp.int32)
    pad_word = (pad_idx * 97 % N) | ((N + pad_idx % (N_PAD - N)) << 16)
    packed1d = jnp.concatenate([packed, pad_word])
    ewp = jnp.concatenate([edge_attr, jnp.zeros((E_PAD - E,), jnp.float32)])
    zeros = jnp.zeros((ROWS_PER_TILE, U), jnp.float32)

    stack1 = jnp.concatenate(
        [x_od, finished_hidden, history, long_his_hidden, yesterday,
         short_his_hidden], axis=0)
    P = _sc_segsum(stack1, 6, packed1d, ewp, zeros)

    X = jnp.stack([x_od, history, yesterday])
    H = jnp.stack([finished_hidden, long_his_hidden, short_his_hidden])
    Wru = jnp.stack([W_ru_fin.reshape(2 * 2 * U, 2 * U),
                     W_ru_long.reshape(2 * 2 * U, 2 * U),
                     W_ru_short.reshape(2 * 2 * U, 2 * U)])
    Wcx = jnp.stack([W_c_fin[:, :U, :].reshape(2 * U, U),
                     W_c_long[:, :U, :].reshape(2 * U, U),
                     W_c_short[:, :U, :].reshape(2 * U, U)])
    Wch = jnp.stack([W_c_fin[:, U:, :].reshape(2 * U, U),
                     W_c_long[:, U:, :].reshape(2 * U, U),
                     W_c_short[:, U:, :].reshape(2 * U, U)])
    Bru = jnp.stack([b_ru_fin, b_ru_long, b_ru_short])[:, None, :]
    Bc = jnp.stack([b_c_fin, b_c_long, b_c_short])[:, None, :]

    RH, Uu, Q = _tc1(X, H, P, Wru, Wcx, Bru, Bc)

    T2 = _sc_segsum(RH.reshape(3 * N, U), 3, packed1d, ewp, zeros)

    hf, hl, hs = _tc2(Uu, Q, RH, T2, H, Wch)

    G = jnp.concatenate([hl, hs], axis=1).reshape(2 * U, N)
    Ml, Ms = _fuse1(G, W_hid, W_out, b_hid.reshape(U, 1), b_out.reshape(U, 1))
    lw = Ml.reshape(N, U)
    sw = Ms.reshape(N, U)

    Eo = _fuse2(hf, hl, hs, lw, sw)
    return (Eo, hf, hl, hs, Eo)


# speedup vs baseline: 1.5807x; 1.0279x over previous
import jax
import jax.numpy as jnp
from jax import lax
from jax.experimental import pallas as pl
from jax.experimental.pallas import tpu as pltpu
from jax.experimental.pallas import tpu_sc as plsc

N = 10000
E = 320000
U = 128
CHUNK = 112
NW = 32
NCHUNK = (E // NW + CHUNK - 1) // CHUNK
EW = NCHUNK * CHUNK
E_PAD = EW * NW
N_PAD = 10112
ROWS_PER_TILE = N_PAD // 16
NBUF = 2
UNR = 8


def _sc_segsum(tbl_stack, T, packed2d, ewb, zeros):
    mesh = plsc.VectorSubcoreMesh(core_axis_name="c", subcore_axis_name="s")
    NQ = NCHUNK // NBUF
    TAIL = NQ * NBUF

    def body(tbl, pk_hbm, ew_hbm, zeros_hbm, out, *refs):
        pk_v, ew_v = refs[0:2]
        srcu = refs[2:2 + NBUF]
        dstu = refs[2 + NBUF:2 + 2 * NBUF]
        rows = refs[2 + 2 * NBUF:2 + 3 * NBUF]
        acc = refs[2 + 3 * NBUF]
        semst = refs[3 + 3 * NBUF]
        semg = refs[4 + 3 * NBUF:4 + 4 * NBUF]
        sems = refs[4 + 4 * NBUF:4 + 5 * NBUF]
        c = lax.axis_index("c")
        s = lax.axis_index("s")
        gw = c * 16 + s

        pltpu.async_copy(pk_hbm.at[pl.ds(gw * EW, EW)], pk_v, semst)
        pltpu.async_copy(ew_hbm.at[pl.ds(gw * EW, EW)], ew_v, semst)
        pltpu.make_async_copy(pk_hbm.at[pl.ds(0, EW)], pk_v, semst).wait()
        pltpu.make_async_copy(ew_hbm.at[pl.ds(0, EW)], ew_v, semst).wait()

        def unpack(t, j, b):
            for g in range(CHUNK // 16):
                sl = pl.ds(g * 16, 16)
                p = pk_v[pl.ds(j * CHUNK + g * 16, 16)]
                srcu[b][sl] = (p & 0xFFFF) + t * N
                dstu[b][sl] = lax.shift_right_logical(p, 16)

        def issue(t, j, b):
            unpack(t, j, b)
            pltpu.async_copy(tbl.at[srcu[b]], rows[b], semg[b])

        def scale(rref, j):
            def grp(g, carry):
                for k in range(UNR):
                    e = g * UNR + k
                    w16 = plsc.load_gather(
                        ew_v, [jnp.broadcast_to(j * CHUNK + e, (16,))])
                    for cg in range(U // 16):
                        sl = pl.ds(cg * 16, 16)
                        rref[e, sl] = rref[e, sl] * w16
                return carry

            lax.fori_loop(0, CHUNK // UNR, grp, 0)

        def one_pass(t, carry):
            pltpu.sync_copy(zeros_hbm, acc.at[pl.ds(s * ROWS_PER_TILE, ROWS_PER_TILE)])
            plsc.subcore_barrier()

            for b in range(NBUF):
                issue(t, b, b)

            def quad_body(jj, carry2):
                j0 = jj * NBUF
                for b in range(NBUF):
                    pltpu.make_async_copy(tbl.at[srcu[b]], rows[b], semg[b]).wait()
                    scale(rows[b], j0 + b)
                    pltpu.async_copy(rows[b], acc.at[dstu[b]], sems[b], add=True)

                @pl.when(jj < NQ - 1)
                def _():
                    for b in range(NBUF):
                        pltpu.make_async_copy(rows[b], acc.at[dstu[b]], sems[b]).wait()
                        issue(t, j0 + b + NBUF, b)

                return carry2

            lax.fori_loop(0, NQ, quad_body, 0)
            for b in range(NBUF):
                pltpu.make_async_copy(rows[b], acc.at[dstu[b]], sems[b]).wait()
            for j in range(TAIL, NCHUNK):
                unpack(t, j, 0)
                pltpu.async_copy(tbl.at[srcu[0]], rows[0], semg[0]).wait()
                scale(rows[0], j)
                pltpu.sync_copy(rows[0], acc.at[dstu[0]], add=True)
            plsc.subcore_barrier()
            pltpu.sync_copy(acc.at[pl.ds(s * ROWS_PER_TILE, ROWS_PER_TILE)],
                            out.at[t, c, pl.ds(s * ROWS_PER_TILE, ROWS_PER_TILE)])
            plsc.subcore_barrier()
            return carry

        lax.fori_loop(0, T, one_pass, 0)

    call = pl.kernel(
        body,
        out_type=jax.ShapeDtypeStruct((T, 2, N_PAD, U), jnp.float32),
        mesh=mesh,
        scratch_types=(
            [pltpu.VMEM((EW,), jnp.int32),
             pltpu.VMEM((EW,), jnp.float32)]
            + [pltpu.VMEM((CHUNK,), jnp.int32) for _ in range(2 * NBUF)]
            + [pltpu.VMEM((CHUNK, U), jnp.float32) for _ in range(NBUF)]
            + [pltpu.VMEM_SHARED((N_PAD, U), jnp.float32)]
            + [pltpu.SemaphoreType.DMA for _ in range(1 + 2 * NBUF)]
        ),
        compiler_params=pltpu.CompilerParams(needs_layout_passes=False),
    )
    return call(tbl_stack, packed2d, ewb, zeros)


BR = 1000


def _tc1_body(X, H, P, Wru, Wcx, Bru, Bc, RHo, Uo, Qo):
    for i in range(3):
        x = X[i]
        h = H[i]
        sx = P[2 * i, 0] + P[2 * i, 1]
        sh = P[2 * i + 1, 0] + P[2 * i + 1, 1]
        cat = jnp.concatenate([x, h, sx, sh], axis=1)
        ru = jax.nn.sigmoid(jnp.dot(cat, Wru[i], preferred_element_type=jnp.float32)
                            + Bru[i])
        r = ru[:, :U]
        u = ru[:, U:]
        RHo[i] = r * h
        Uo[i] = u
        catq = jnp.concatenate([x, sx], axis=1)
        Qo[i] = jnp.dot(catq, Wcx[i], preferred_element_type=jnp.float32) + Bc[i]


def _tc1(X, H, P, Wru, Wcx, Bru, Bc):
    nb = N // BR
    blk_cell = pl.BlockSpec((3, BR, U), lambda i: (0, i, 0))
    out_row = pl.BlockSpec((BR, U), lambda i: (i, 0))
    return pl.pallas_call(
        _tc1_body,
        grid=(nb,),
        in_specs=[
            blk_cell,
            blk_cell,
            pl.BlockSpec((6, 2, BR, U), lambda i: (0, 0, i, 0)),
            pl.BlockSpec((3, 4 * U, 2 * U), lambda i: (0, 0, 0)),
            pl.BlockSpec((3, 2 * U, U), lambda i: (0, 0, 0)),
            pl.BlockSpec((3, 1, 2 * U), lambda i: (0, 0, 0)),
            pl.BlockSpec((3, 1, U), lambda i: (0, 0, 0)),
        ],
        out_specs=[blk_cell, blk_cell, blk_cell],
        out_shape=[
            jax.ShapeDtypeStruct((3, N, U), jnp.float32),
            jax.ShapeDtypeStruct((3, N, U), jnp.float32),
            jax.ShapeDtypeStruct((3, N, U), jnp.float32),
        ],
    )(X, H, P, Wru, Wcx, Bru, Bc)


def _tc2_body(Uu, Q, RH, T2, H, Wch, hf, hl, hs):
    outs = (hf, hl, hs)
    for i in range(3):
        t = T2[i, 0] + T2[i, 1]
        cat = jnp.concatenate([RH[i], t], axis=1)
        c = jnp.tanh(Q[i] + jnp.dot(cat, Wch[i], preferred_element_type=jnp.float32))
        u = Uu[i]
        outs[i][...] = u * H[i] + (1.0 - u) * c


def _tc2(Uu, Q, RH, T2, H, Wch):
    nb = N // BR
    blk_cell = pl.BlockSpec((3, BR, U), lambda i: (0, i, 0))
    out_row = pl.BlockSpec((BR, U), lambda i: (i, 0))
    return pl.pallas_call(
        _tc2_body,
        grid=(nb,),
        in_specs=[
            blk_cell,
            blk_cell,
            blk_cell,
            pl.BlockSpec((3, 2, BR, U), lambda i: (0, 0, i, 0)),
            blk_cell,
            pl.BlockSpec((3, 2 * U, U), lambda i: (0, 0, 0)),
        ],
        out_specs=[out_row, out_row, out_row],
        out_shape=[jax.ShapeDtypeStruct((N, U), jnp.float32)] * 3,
    )(Uu, Q, RH, T2, H, Wch)


def _fuse1_body(G, Wh, Wo, bh, bo, Ml, Ms):
    g = G[...]
    Ml[...] = jax.nn.sigmoid(jnp.dot(Wh[...], g, preferred_element_type=jnp.float32) + bh[...])
    Ms[...] = jax.nn.sigmoid(jnp.dot(Wo[...], g, preferred_element_type=jnp.float32) + bo[...])


def _fuse1(G, Wh, Wo, bh, bo):
    return pl.pallas_call(
        _fuse1_body,
        out_shape=[jax.ShapeDtypeStruct((U, N), jnp.float32)] * 2,
    )(G, Wh, Wo, bh, bo)


def _fuse2_body(hf, hl, hs, lw, sw, Eo):
    Eo[...] = hf[...] + lw[...] * hl[...] + sw[...] * hs[...]


def _fuse2(hf, hl, hs, lw, sw):
    nb = N // BR
    row = pl.BlockSpec((BR, U), lambda i: (i, 0))
    return pl.pallas_call(
        _fuse2_body,
        grid=(nb,),
        in_specs=[row] * 5,
        out_specs=row,
        out_shape=jax.ShapeDtypeStruct((N, U), jnp.float32),
    )(hf, hl, hs, lw, sw)


def kernel(x_od, history, yesterday, finished_hidden, long_his_hidden,
           short_his_hidden, edge_index, edge_attr, W_ru_fin, b_ru_fin,
           W_c_fin, b_c_fin, W_ru_long, b_ru_long, W_c_long, b_c_long,
           W_ru_short, b_ru_short, W_c_short, b_c_short, W_hid, b_hid,
           W_out, b_out):
    packed = edge_index[0] | (edge_index[1] << 16)
    pad_idx = jnp.arange(E_PAD - E, dtype=jnp.int32)
    pad_word = (pad_idx * 97 % N) | ((N + pad_idx % (N_PAD - N)) << 16)
    packed1d = jnp.concatenate([packed, pad_word])
    ewp = jnp.concatenate([edge_attr, jnp.zeros((E_PAD - E,), jnp.float32)])
    zeros = jnp.zeros((ROWS_PER_TILE, U), jnp.float32)

    stack1 = jnp.concatenate(
        [x_od, finished_hidden, history, long_his_hidden, yesterday,
         short_his_hidden], axis=0)
    P = _sc_segsum(stack1, 6, packed1d, ewp, zeros)

    X = jnp.stack([x_od, history, yesterday])
    H = jnp.stack([finished_hidden, long_his_hidden, short_his_hidden])
    Wru = jnp.stack([W_ru_fin.reshape(2 * 2 * U, 2 * U),
                     W_ru_long.reshape(2 * 2 * U, 2 * U),
                     W_ru_short.reshape(2 * 2 * U, 2 * U)])
    Wcx = jnp.stack([W_c_fin[:, :U, :].reshape(2 * U, U),
                     W_c_long[:, :U, :].reshape(2 * U, U),
                     W_c_short[:, :U, :].reshape(2 * U, U)])
    Wch = jnp.stack([W_c_fin[:, U:, :].reshape(2 * U, U),
                     W_c_long[:, U:, :].reshape(2 * U, U),
                     W_c_short[:, U:, :].reshape(2 * U, U)])
    Bru = jnp.stack([b_ru_fin, b_ru_long, b_ru_short])[:, None, :]
    Bc = jnp.stack([b_c_fin, b_c_long, b_c_short])[:, None, :]

    RH, Uu, Q = _tc1(X, H, P, Wru, Wcx, Bru, Bc)

    T2 = _sc_segsum(RH.reshape(3 * N, U), 3, packed1d, ewp, zeros)

    hf, hl, hs = _tc2(Uu, Q, RH, T2, H, Wch)

    G = jnp.concatenate([hl, hs], axis=1).reshape(2 * U, N)
    Ml, Ms = _fuse1(G, W_hid, W_out, b_hid.reshape(U, 1), b_out.reshape(U, 1))
    lw = Ml.reshape(N, U)
    sw = Ms.reshape(N, U)

    Eo = _fuse2(hf, hl, hs, lw, sw)
    return (Eo, hf, hl, hs, Eo)


# speedup vs baseline: 1.5904x; 1.0061x over previous
import jax
import jax.numpy as jnp
from jax import lax
from jax.experimental import pallas as pl
from jax.experimental.pallas import tpu as pltpu
from jax.experimental.pallas import tpu_sc as plsc

N = 10000
E = 320000
U = 128
CHUNK = 128
NW = 32
NCHUNK = (E // NW + CHUNK - 1) // CHUNK
EW = NCHUNK * CHUNK
E_PAD = EW * NW
N_PAD = 10112
ROWS_PER_TILE = N_PAD // 16
NBUF = 2
UNR = 8


def _sc_segsum(tbl_stack, T, packed2d, ewb, zeros):
    mesh = plsc.VectorSubcoreMesh(core_axis_name="c", subcore_axis_name="s")
    NQ = NCHUNK // NBUF
    TAIL = NQ * NBUF

    def body(tbl, pk_hbm, ew_hbm, zeros_hbm, out, *refs):
        pk_v = refs[0]
        srcu = refs[1:1 + NBUF]
        dstu = refs[1 + NBUF:1 + 2 * NBUF]
        rows = refs[1 + 2 * NBUF:1 + 3 * NBUF]
        ewc = refs[1 + 3 * NBUF:1 + 4 * NBUF]
        acc = refs[1 + 4 * NBUF]
        semst = refs[2 + 4 * NBUF]
        semg = refs[3 + 4 * NBUF:3 + 5 * NBUF]
        sems = refs[3 + 5 * NBUF:3 + 6 * NBUF]
        c = lax.axis_index("c")
        s = lax.axis_index("s")
        gw = c * 16 + s

        pltpu.async_copy(pk_hbm.at[pl.ds(gw * EW, EW)], pk_v, semst)
        pltpu.make_async_copy(pk_hbm.at[pl.ds(0, EW)], pk_v, semst).wait()

        def unpack(t, j, b):
            for g in range(CHUNK // 16):
                sl = pl.ds(g * 16, 16)
                p = pk_v[pl.ds(j * CHUNK + g * 16, 16)]
                srcu[b][sl] = (p & 0xFFFF) + t * N
                dstu[b][sl] = lax.shift_right_logical(p, 16)

        def issue(t, j, b):
            unpack(t, j, b)
            pltpu.async_copy(tbl.at[srcu[b]], rows[b], semg[b])
            pltpu.async_copy(ew_hbm.at[pl.ds(gw * EW + j * CHUNK, CHUNK)],
                             ewc[b], semg[b])

        def scale(rref, eref):
            def grp(g, carry):
                for k in range(UNR):
                    e = g * UNR + k
                    w16 = plsc.load_gather(eref, [jnp.broadcast_to(e, (16,))])
                    for cg in range(U // 16):
                        sl = pl.ds(cg * 16, 16)
                        rref[e, sl] = rref[e, sl] * w16
                return carry

            lax.fori_loop(0, CHUNK // UNR, grp, 0)

        def one_pass(t, carry):
            pltpu.sync_copy(zeros_hbm, acc.at[pl.ds(s * ROWS_PER_TILE, ROWS_PER_TILE)])
            plsc.subcore_barrier()

            for b in range(NBUF):
                issue(t, b, b)

            def quad_body(jj, carry2):
                j0 = jj * NBUF
                for b in range(NBUF):
                    pltpu.make_async_copy(tbl.at[srcu[b]], rows[b], semg[b]).wait()
                    pltpu.make_async_copy(ew_hbm.at[pl.ds(0, CHUNK)], ewc[b],
                                          semg[b]).wait()
                    scale(rows[b], ewc[b])
                    pltpu.async_copy(rows[b], acc.at[dstu[b]], sems[b], add=True)

                @pl.when(jj < NQ - 1)
                def _():
                    for b in range(NBUF):
                        pltpu.make_async_copy(rows[b], acc.at[dstu[b]], sems[b]).wait()
                        issue(t, j0 + b + NBUF, b)

                return carry2

            lax.fori_loop(0, NQ, quad_body, 0)
            for b in range(NBUF):
                pltpu.make_async_copy(rows[b], acc.at[dstu[b]], sems[b]).wait()
            for j in range(TAIL, NCHUNK):
                unpack(t, j, 0)
                pltpu.sync_copy(ew_hbm.at[pl.ds(gw * EW + j * CHUNK, CHUNK)], ewc[0])
                pltpu.async_copy(tbl.at[srcu[0]], rows[0], semg[0]).wait()
                scale(rows[0], ewc[0])
                pltpu.sync_copy(rows[0], acc.at[dstu[0]], add=True)
            plsc.subcore_barrier()
            pltpu.sync_copy(acc.at[pl.ds(s * ROWS_PER_TILE, ROWS_PER_TILE)],
                            out.at[t, c, pl.ds(s * ROWS_PER_TILE, ROWS_PER_TILE)])
            plsc.subcore_barrier()
            return carry

        lax.fori_loop(0, T, one_pass, 0)

    call = pl.kernel(
        body,
        out_type=jax.ShapeDtypeStruct((T, 2, N_PAD, U), jnp.float32),
        mesh=mesh,
        scratch_types=(
            [pltpu.VMEM((EW,), jnp.int32)]
            + [pltpu.VMEM((CHUNK,), jnp.int32) for _ in range(2 * NBUF)]
            + [pltpu.VMEM((CHUNK, U), jnp.float32) for _ in range(NBUF)]
            + [pltpu.VMEM((CHUNK,), jnp.float32) for _ in range(NBUF)]
            + [pltpu.VMEM_SHARED((N_PAD, U), jnp.float32)]
            + [pltpu.SemaphoreType.DMA for _ in range(1 + 2 * NBUF)]
        ),
        compiler_params=pltpu.CompilerParams(needs_layout_passes=False),
    )
    return call(tbl_stack, packed2d, ewb, zeros)


BR = 1000


def _tc1_body(X, H, P, Wru, Wcx, Bru, Bc, RHo, Uo, Qo):
    for i in range(3):
        x = X[i]
        h = H[i]
        sx = P[2 * i, 0] + P[2 * i, 1]
        sh = P[2 * i + 1, 0] + P[2 * i + 1, 1]
        cat = jnp.concatenate([x, h, sx, sh], axis=1)
        ru = jax.nn.sigmoid(jnp.dot(cat, Wru[i], preferred_element_type=jnp.float32)
                            + Bru[i])
        r = ru[:, :U]
        u = ru[:, U:]
        RHo[i] = r * h
        Uo[i] = u
        catq = jnp.concatenate([x, sx], axis=1)
        Qo[i] = jnp.dot(catq, Wcx[i], preferred_element_type=jnp.float32) + Bc[i]


def _tc1(X, H, P, Wru, Wcx, Bru, Bc):
    nb = N // BR
    blk_cell = pl.BlockSpec((3, BR, U), lambda i: (0, i, 0))
    out_row = pl.BlockSpec((BR, U), lambda i: (i, 0))
    return pl.pallas_call(
        _tc1_body,
        grid=(nb,),
        in_specs=[
            blk_cell,
            blk_cell,
            pl.BlockSpec((6, 2, BR, U), lambda i: (0, 0, i, 0)),
            pl.BlockSpec((3, 4 * U, 2 * U), lambda i: (0, 0, 0)),
            pl.BlockSpec((3, 2 * U, U), lambda i: (0, 0, 0)),
            pl.BlockSpec((3, 1, 2 * U), lambda i: (0, 0, 0)),
            pl.BlockSpec((3, 1, U), lambda i: (0, 0, 0)),
        ],
        out_specs=[blk_cell, blk_cell, blk_cell],
        out_shape=[
            jax.ShapeDtypeStruct((3, N, U), jnp.float32),
            jax.ShapeDtypeStruct((3, N, U), jnp.float32),
            jax.ShapeDtypeStruct((3, N, U), jnp.float32),
        ],
    )(X, H, P, Wru, Wcx, Bru, Bc)


def _tc2_body(Uu, Q, RH, T2, H, Wch, hf, hl, hs):
    outs = (hf, hl, hs)
    for i in range(3):
        t = T2[i, 0] + T2[i, 1]
        cat = jnp.concatenate([RH[i], t], axis=1)
        c = jnp.tanh(Q[i] + jnp.dot(cat, Wch[i], preferred_element_type=jnp.float32))
        u = Uu[i]
        outs[i][...] = u * H[i] + (1.0 - u) * c


def _tc2(Uu, Q, RH, T2, H, Wch):
    nb = N // BR
    blk_cell = pl.BlockSpec((3, BR, U), lambda i: (0, i, 0))
    out_row = pl.BlockSpec((BR, U), lambda i: (i, 0))
    return pl.pallas_call(
        _tc2_body,
        grid=(nb,),
        in_specs=[
            blk_cell,
            blk_cell,
            blk_cell,
            pl.BlockSpec((3, 2, BR, U), lambda i: (0, 0, i, 0)),
            blk_cell,
            pl.BlockSpec((3, 2 * U, U), lambda i: (0, 0, 0)),
        ],
        out_specs=[out_row, out_row, out_row],
        out_shape=[jax.ShapeDtypeStruct((N, U), jnp.float32)] * 3,
    )(Uu, Q, RH, T2, H, Wch)


def _fuse1_body(G, Wh, Wo, bh, bo, Ml, Ms):
    g = G[...]
    Ml[...] = jax.nn.sigmoid(jnp.dot(Wh[...], g, preferred_element_type=jnp.float32) + bh[...])
    Ms[...] = jax.nn.sigmoid(jnp.dot(Wo[...], g, preferred_element_type=jnp.float32) + bo[...])


def _fuse1(G, Wh, Wo, bh, bo):
    return pl.pallas_call(
        _fuse1_body,
        out_shape=[jax.ShapeDtypeStruct((U, N), jnp.float32)] * 2,
    )(G, Wh, Wo, bh, bo)


def _fuse2_body(hf, hl, hs, lw, sw, Eo):
    Eo[...] = hf[...] + lw[...] * hl[...] + sw[...] * hs[...]


def _fuse2(hf, hl, hs, lw, sw):
    nb = N // BR
    row = pl.BlockSpec((BR, U), lambda i: (i, 0))
    return pl.pallas_call(
        _fuse2_body,
        grid=(nb,),
        in_specs=[row] * 5,
        out_specs=row,
        out_shape=jax.ShapeDtypeStruct((N, U), jnp.float32),
    )(hf, hl, hs, lw, sw)


def kernel(x_od, history, yesterday, finished_hidden, long_his_hidden,
           short_his_hidden, edge_index, edge_attr, W_ru_fin, b_ru_fin,
           W_c_fin, b_c_fin, W_ru_long, b_ru_long, W_c_long, b_c_long,
           W_ru_short, b_ru_short, W_c_short, b_c_short, W_hid, b_hid,
           W_out, b_out):
    packed = edge_index[0] | (edge_index[1] << 16)
    pad_idx = jnp.arange(E_PAD - E, dtype=jnp.int32)
    pad_word = (pad_idx * 97 % N) | ((N + pad_idx % (N_PAD - N)) << 16)
    packed1d = jnp.concatenate([packed, pad_word])
    ewp = jnp.concatenate([edge_attr, jnp.zeros((E_PAD - E,), jnp.float32)])
    zeros = jnp.zeros((ROWS_PER_TILE, U), jnp.float32)

    stack1 = jnp.concatenate(
        [x_od, finished_hidden, history, long_his_hidden, yesterday,
         short_his_hidden], axis=0)
    P = _sc_segsum(stack1, 6, packed1d, ewp, zeros)

    X = jnp.stack([x_od, history, yesterday])
    H = jnp.stack([finished_hidden, long_his_hidden, short_his_hidden])
    Wru = jnp.stack([W_ru_fin.reshape(2 * 2 * U, 2 * U),
                     W_ru_long.reshape(2 * 2 * U, 2 * U),
                     W_ru_short.reshape(2 * 2 * U, 2 * U)])
    Wcx = jnp.stack([W_c_fin[:, :U, :].reshape(2 * U, U),
                     W_c_long[:, :U, :].reshape(2 * U, U),
                     W_c_short[:, :U, :].reshape(2 * U, U)])
    Wch = jnp.stack([W_c_fin[:, U:, :].reshape(2 * U, U),
                     W_c_long[:, U:, :].reshape(2 * U, U),
                     W_c_short[:, U:, :].reshape(2 * U, U)])
    Bru = jnp.stack([b_ru_fin, b_ru_long, b_ru_short])[:, None, :]
    Bc = jnp.stack([b_c_fin, b_c_long, b_c_short])[:, None, :]

    RH, Uu, Q = _tc1(X, H, P, Wru, Wcx, Bru, Bc)

    T2 = _sc_segsum(RH.reshape(3 * N, U), 3, packed1d, ewp, zeros)

    hf, hl, hs = _tc2(Uu, Q, RH, T2, H, Wch)

    G = jnp.concatenate([hl, hs], axis=1).reshape(2 * U, N)
    Ml, Ms = _fuse1(G, W_hid, W_out, b_hid.reshape(U, 1), b_out.reshape(U, 1))
    lw = Ml.reshape(N, U)
    sw = Ms.reshape(N, U)

    Eo = _fuse2(hf, hl, hs, lw, sw)
    return (Eo, hf, hl, hs, Eo)


# speedup vs baseline: 1.5938x; 1.0021x over previous
import jax
import jax.numpy as jnp
from jax import lax
from jax.experimental import pallas as pl
from jax.experimental.pallas import tpu as pltpu
from jax.experimental.pallas import tpu_sc as plsc

N = 10000
E = 320000
U = 128
CHUNK = 128
NW = 32
NCHUNK = (E // NW + CHUNK - 1) // CHUNK
EW = NCHUNK * CHUNK
E_PAD = EW * NW
N_PAD = 10112
ROWS_PER_TILE = N_PAD // 16
NBUF = 2
UNR = 8


def _sc_segsum(tbl_stack, T, packed2d, ewb, zeros):
    mesh = plsc.VectorSubcoreMesh(core_axis_name="c", subcore_axis_name="s")
    NQ = NCHUNK // NBUF
    TAIL = NQ * NBUF

    def body(tbl, pk_hbm, ew_hbm, zeros_hbm, out, *refs):
        pk_v = refs[0]
        srcu = refs[1:1 + NBUF]
        dstu = refs[1 + NBUF:1 + 2 * NBUF]
        rows = refs[1 + 2 * NBUF:1 + 3 * NBUF]
        ewc = refs[1 + 3 * NBUF:1 + 4 * NBUF]
        acc = refs[1 + 4 * NBUF]
        semst = refs[2 + 4 * NBUF]
        semg = refs[3 + 4 * NBUF:3 + 5 * NBUF]
        sems = refs[3 + 5 * NBUF:3 + 6 * NBUF]
        c = lax.axis_index("c")
        s = lax.axis_index("s")
        gw = c * 16 + s

        pltpu.async_copy(pk_hbm.at[pl.ds(gw * EW, EW)], pk_v, semst)
        pltpu.make_async_copy(pk_hbm.at[pl.ds(0, EW)], pk_v, semst).wait()

        def unpack(t, j, b):
            for g in range(CHUNK // 16):
                sl = pl.ds(g * 16, 16)
                p = pk_v[pl.ds(j * CHUNK + g * 16, 16)]
                srcu[b][sl] = (p & 0xFFFF) + t * N
                dstu[b][sl] = lax.shift_right_logical(p, 16)

        def issue(t, j, b):
            unpack(t, j, b)
            pltpu.async_copy(tbl.at[srcu[b]], rows[b], semg[b])
            pltpu.async_copy(ew_hbm.at[pl.ds(gw * EW + j * CHUNK, CHUNK)],
                             ewc[b], semg[b])

        def scale(rref, eref):
            def grp(g, carry):
                for k in range(UNR):
                    e = g * UNR + k
                    w16 = plsc.load_gather(eref, [jnp.broadcast_to(e, (16,))])
                    for cg in range(U // 16):
                        sl = pl.ds(cg * 16, 16)
                        rref[e, sl] = rref[e, sl] * w16
                return carry

            lax.fori_loop(0, CHUNK // UNR, grp, 0)

        def one_pass(t, carry):
            pltpu.sync_copy(zeros_hbm.at[s], acc.at[pl.ds(s * ROWS_PER_TILE, ROWS_PER_TILE)])
            plsc.subcore_barrier()

            for b in range(NBUF):
                issue(t, b, b)

            def quad_body(jj, carry2):
                j0 = jj * NBUF
                for b in range(NBUF):
                    pltpu.make_async_copy(tbl.at[srcu[b]], rows[b], semg[b]).wait()
                    pltpu.make_async_copy(ew_hbm.at[pl.ds(0, CHUNK)], ewc[b],
                                          semg[b]).wait()
                    scale(rows[b], ewc[b])
                    pltpu.async_copy(rows[b], acc.at[dstu[b]], sems[b], add=True)

                @pl.when(jj < NQ - 1)
                def _():
                    for b in range(NBUF):
                        pltpu.make_async_copy(rows[b], acc.at[dstu[b]], sems[b]).wait()
                        issue(t, j0 + b + NBUF, b)

                return carry2

            lax.fori_loop(0, NQ, quad_body, 0)
            for b in range(NBUF):
                pltpu.make_async_copy(rows[b], acc.at[dstu[b]], sems[b]).wait()
            for j in range(TAIL, NCHUNK):
                unpack(t, j, 0)
                pltpu.sync_copy(ew_hbm.at[pl.ds(gw * EW + j * CHUNK, CHUNK)], ewc[0])
                pltpu.async_copy(tbl.at[srcu[0]], rows[0], semg[0]).wait()
                scale(rows[0], ewc[0])
                pltpu.sync_copy(rows[0], acc.at[dstu[0]], add=True)
            plsc.subcore_barrier()
            pltpu.sync_copy(acc.at[pl.ds(s * ROWS_PER_TILE, ROWS_PER_TILE)],
                            out.at[t, c, pl.ds(s * ROWS_PER_TILE, ROWS_PER_TILE)])
            plsc.subcore_barrier()
            return carry

        lax.fori_loop(0, T, one_pass, 0)

    call = pl.kernel(
        body,
        out_type=jax.ShapeDtypeStruct((T, 2, N_PAD, U), jnp.float32),
        mesh=mesh,
        scratch_types=(
            [pltpu.VMEM((EW,), jnp.int32)]
            + [pltpu.VMEM((CHUNK,), jnp.int32) for _ in range(2 * NBUF)]
            + [pltpu.VMEM((CHUNK, U), jnp.float32) for _ in range(NBUF)]
            + [pltpu.VMEM((CHUNK,), jnp.float32) for _ in range(NBUF)]
            + [pltpu.VMEM_SHARED((N_PAD, U), jnp.float32)]
            + [pltpu.SemaphoreType.DMA for _ in range(1 + 2 * NBUF)]
        ),
        compiler_params=pltpu.CompilerParams(needs_layout_passes=False),
    )
    return call(tbl_stack, packed2d, ewb, zeros)


BR = 1000


def _tc1_body(X, H, P, Wru, Wcx, Bru, Bc, RHo, Uo, Qo):
    for i in range(3):
        x = X[i]
        h = H[i]
        sx = P[2 * i, 0] + P[2 * i, 1]
        sh = P[2 * i + 1, 0] + P[2 * i + 1, 1]
        cat = jnp.concatenate([x, h, sx, sh], axis=1)
        ru = jax.nn.sigmoid(jnp.dot(cat, Wru[i], preferred_element_type=jnp.float32)
                            + Bru[i])
        r = ru[:, :U]
        u = ru[:, U:]
        RHo[i] = r * h
        Uo[i] = u
        catq = jnp.concatenate([x, sx], axis=1)
        Qo[i] = jnp.dot(catq, Wcx[i], preferred_element_type=jnp.float32) + Bc[i]


def _tc1(X, H, P, Wru, Wcx, Bru, Bc):
    nb = N // BR
    blk_cell = pl.BlockSpec((3, BR, U), lambda i: (0, i, 0))
    out_row = pl.BlockSpec((BR, U), lambda i: (i, 0))
    return pl.pallas_call(
        _tc1_body,
        grid=(nb,),
        in_specs=[
            blk_cell,
            blk_cell,
            pl.BlockSpec((6, 2, BR, U), lambda i: (0, 0, i, 0)),
            pl.BlockSpec((3, 4 * U, 2 * U), lambda i: (0, 0, 0)),
            pl.BlockSpec((3, 2 * U, U), lambda i: (0, 0, 0)),
            pl.BlockSpec((3, 1, 2 * U), lambda i: (0, 0, 0)),
            pl.BlockSpec((3, 1, U), lambda i: (0, 0, 0)),
        ],
        out_specs=[blk_cell, blk_cell, blk_cell],
        out_shape=[
            jax.ShapeDtypeStruct((3, N, U), jnp.float32),
            jax.ShapeDtypeStruct((3, N, U), jnp.float32),
            jax.ShapeDtypeStruct((3, N, U), jnp.float32),
        ],
    )(X, H, P, Wru, Wcx, Bru, Bc)


def _tc2_body(Uu, Q, RH, T2, H, Wch, hf, hl, hs):
    outs = (hf, hl, hs)
    for i in range(3):
        t = T2[i, 0] + T2[i, 1]
        cat = jnp.concatenate([RH[i], t], axis=1)
        c = jnp.tanh(Q[i] + jnp.dot(cat, Wch[i], preferred_element_type=jnp.float32))
        u = Uu[i]
        outs[i][...] = u * H[i] + (1.0 - u) * c


def _tc2(Uu, Q, RH, T2, H, Wch):
    nb = N // BR
    blk_cell = pl.BlockSpec((3, BR, U), lambda i: (0, i, 0))
    out_row = pl.BlockSpec((BR, U), lambda i: (i, 0))
    return pl.pallas_call(
        _tc2_body,
        grid=(nb,),
        in_specs=[
            blk_cell,
            blk_cell,
            blk_cell,
            pl.BlockSpec((3, 2, BR, U), lambda i: (0, 0, i, 0)),
            blk_cell,
            pl.BlockSpec((3, 2 * U, U), lambda i: (0, 0, 0)),
        ],
        out_specs=[out_row, out_row, out_row],
        out_shape=[jax.ShapeDtypeStruct((N, U), jnp.float32)] * 3,
    )(Uu, Q, RH, T2, H, Wch)


def _fuse1_body(G, Wh, Wo, bh, bo, Ml, Ms):
    g = G[...]
    Ml[...] = jax.nn.sigmoid(jnp.dot(Wh[...], g, preferred_element_type=jnp.float32) + bh[...])
    Ms[...] = jax.nn.sigmoid(jnp.dot(Wo[...], g, preferred_element_type=jnp.float32) + bo[...])


def _fuse1(G, Wh, Wo, bh, bo):
    return pl.pallas_call(
        _fuse1_body,
        out_shape=[jax.ShapeDtypeStruct((U, N), jnp.float32)] * 2,
    )(G, Wh, Wo, bh, bo)


def _fuse2_body(hf, hl, hs, lw, sw, Eo):
    Eo[...] = hf[...] + lw[...] * hl[...] + sw[...] * hs[...]


def _fuse2(hf, hl, hs, lw, sw):
    nb = N // BR
    row = pl.BlockSpec((BR, U), lambda i: (i, 0))
    return pl.pallas_call(
        _fuse2_body,
        grid=(nb,),
        in_specs=[row] * 5,
        out_specs=row,
        out_shape=jax.ShapeDtypeStruct((N, U), jnp.float32),
    )(hf, hl, hs, lw, sw)


def kernel(x_od, history, yesterday, finished_hidden, long_his_hidden,
           short_his_hidden, edge_index, edge_attr, W_ru_fin, b_ru_fin,
           W_c_fin, b_c_fin, W_ru_long, b_ru_long, W_c_long, b_c_long,
           W_ru_short, b_ru_short, W_c_short, b_c_short, W_hid, b_hid,
           W_out, b_out):
    packed = edge_index[0] | (edge_index[1] << 16)
    pad_idx = jnp.arange(E_PAD - E, dtype=jnp.int32)
    pad_word = (pad_idx * 97 % N) | ((N + pad_idx % (N_PAD - N)) << 16)
    packed1d = jnp.concatenate([packed, pad_word])
    ewp = jnp.concatenate([edge_attr, jnp.zeros((E_PAD - E,), jnp.float32)])
    zeros = jnp.zeros((16, ROWS_PER_TILE, U), jnp.float32)

    stack1 = jnp.concatenate(
        [x_od, finished_hidden, history, long_his_hidden, yesterday,
         short_his_hidden], axis=0)
    P = _sc_segsum(stack1, 6, packed1d, ewp, zeros)

    X = jnp.stack([x_od, history, yesterday])
    H = jnp.stack([finished_hidden, long_his_hidden, short_his_hidden])
    Wru = jnp.stack([W_ru_fin.reshape(2 * 2 * U, 2 * U),
                     W_ru_long.reshape(2 * 2 * U, 2 * U),
                     W_ru_short.reshape(2 * 2 * U, 2 * U)])
    Wcx = jnp.stack([W_c_fin[:, :U, :].reshape(2 * U, U),
                     W_c_long[:, :U, :].reshape(2 * U, U),
                     W_c_short[:, :U, :].reshape(2 * U, U)])
    Wch = jnp.stack([W_c_fin[:, U:, :].reshape(2 * U, U),
                     W_c_long[:, U:, :].reshape(2 * U, U),
                     W_c_short[:, U:, :].reshape(2 * U, U)])
    Bru = jnp.stack([b_ru_fin, b_ru_long, b_ru_short])[:, None, :]
    Bc = jnp.stack([b_c_fin, b_c_long, b_c_short])[:, None, :]

    RH, Uu, Q = _tc1(X, H, P, Wru, Wcx, Bru, Bc)

    T2 = _sc_segsum(RH.reshape(3 * N, U), 3, packed1d, ewp, zeros)

    hf, hl, hs = _tc2(Uu, Q, RH, T2, H, Wch)

    G = jnp.concatenate([hl, hs], axis=1).reshape(2 * U, N)
    Ml, Ms = _fuse1(G, W_hid, W_out, b_hid.reshape(U, 1), b_out.reshape(U, 1))
    lw = Ml.reshape(N, U)
    sw = Ms.reshape(N, U)

    Eo = _fuse2(hf, hl, hs, lw, sw)
    return (Eo, hf, hl, hs, Eo)
